# Initial kernel scaffold; baseline (speedup 1.0000x reference)
import jax
import jax.numpy as jnp
from jax.experimental import pallas as pl


def kernel(x, edge_index, edge_attr, batch, nfc_w, nfc_b, efc1_w1, efc1_b1, efc1_w2, efc1_b2, gc1_root, gc1_bias, efc2_w1, efc2_b1, efc2_w2, efc2_b2, gc2_root, gc2_bias, fc_w, fc_b):
    raise NotImplementedError("write your pallas kernel here")


# speedup vs baseline: 3.5684x; 3.5684x over previous
import functools
import jax
import jax.numpy as jnp
import numpy as np
from jax import lax
from jax.experimental import pallas as pl
from jax.experimental.pallas import tpu as pltpu
from jax.experimental.pallas import tpu_sc as plsc

_N, _E, _DN, _DE, _H, _DO, _G = 10000, 160000, 128, 16, 16, 64, 256

_NC, _NS = 2, 16
_NW = _NC * _NS
_CH = 128
_EPW = 5120
_E_PAD = _EPW * _NW
_NCHUNK = _EPW // _CH
_N_PAD = 10240
_RPS = _N_PAD // _NS


def _lrelu(v):
    return jnp.where(v >= 0, v, 0.01 * v)


def _h0_body(x_ref, w_ref, b_ref, o_ref):
    t = jnp.dot(x_ref[...], w_ref[...], preferred_element_type=jnp.float32)
    o_ref[...] = _lrelu(t + b_ref[...])


def _h0(x, w, b):
    return pl.pallas_call(
        _h0_body,
        out_shape=jax.ShapeDtypeStruct((_N, _H), jnp.float32),
    )(x, w, b)


def _msg_body(ea_ref, hs_ref, w1_ref, b1_ref, v_ref, b2m_ref, p_ref, q_ref,
              o_ref):
    a = jnp.maximum(
        jnp.dot(ea_ref[...], w1_ref[...], preferred_element_type=jnp.float32)
        + b1_ref[...], 0.0)
    bt = (jnp.dot(a, p_ref[...], preferred_element_type=jnp.float32)
          * jnp.dot(hs_ref[...], q_ref[...], preferred_element_type=jnp.float32))
    o_ref[...] = (
        jnp.dot(bt, v_ref[...], preferred_element_type=jnp.float32)
        + jnp.dot(hs_ref[...], b2m_ref[...], preferred_element_type=jnp.float32))


def _msg(ea_p, hs, w1, b1, v, b2m, p, q, eb=4096):
    nblk = _E_PAD // eb
    full = lambda s: pl.BlockSpec(s, lambda i: (0, 0))
    return pl.pallas_call(
        _msg_body,
        grid=(nblk,),
        in_specs=[
            pl.BlockSpec((eb, _DE), lambda i: (i, 0)),
            pl.BlockSpec((eb, _H), lambda i: (i, 0)),
            full((_DE, _H)), full((1, _H)), full((_H * _H, _H)),
            full((_H, _H)), full((_H, _H * _H)), full((_H, _H * _H)),
        ],
        out_specs=pl.BlockSpec((eb, _H), lambda i: (i, 0)),
        out_shape=jax.ShapeDtypeStruct((_E_PAD, _H), jnp.float32),
        compiler_params=pltpu.CompilerParams(
            dimension_semantics=("arbitrary",)),
    )(ea_p, hs, w1, b1, v, b2m, p, q)


def _upd_body(p0_ref, p1_ref, h_ref, r_ref, b_ref, o_ref):
    t = (p0_ref[...] + p1_ref[...]
         + jnp.dot(h_ref[...], r_ref[...], preferred_element_type=jnp.float32)
         + b_ref[...])
    o_ref[...] = _lrelu(t)


def _upd(p0, p1, h, r, b):
    return pl.pallas_call(
        _upd_body,
        out_shape=jax.ShapeDtypeStruct((_N, _H), jnp.float32),
    )(p0, p1, h, r, b)


def _final_body(p0_ref, p1_ref, h_ref, batch_ref, r_ref, b_ref, fcw_ref,
                fcb_ref, ae_ref, out_ref, acc_ref):
    i = pl.program_id(0)
    nblk = pl.num_programs(0)
    t = (p0_ref[...] + p1_ref[...]
         + jnp.dot(h_ref[...], r_ref[...], preferred_element_type=jnp.float32)
         + b_ref[...])
    ae = _lrelu(t)
    ae_ref[...] = ae
    bvec = batch_ref[0, 0, :]
    onehot_t = (lax.broadcasted_iota(jnp.int32, (_G, bvec.shape[0]), 0)
                == bvec[None, :]).astype(jnp.float32)
    part = jnp.dot(onehot_t, ae, preferred_element_type=jnp.float32)

    @pl.when(i == 0)
    def _():
        acc_ref[...] = part

    @pl.when(i > 0)
    def _():
        acc_ref[...] = acc_ref[...] + part

    @pl.when(i == nblk - 1)
    def _():
        pooled = acc_ref[...]
        nrm = jnp.sqrt(jnp.sum(pooled * pooled, axis=1, keepdims=True))
        normed = pooled / jnp.maximum(nrm, 1e-12)
        out_ref[...] = (
            jnp.dot(normed, fcw_ref[...], preferred_element_type=jnp.float32)
            + fcb_ref[...])


def _final(p0, p1, h, batch3, r, b, fcw, fcb, nb=1000):
    nblk = _N // nb
    full = lambda s: pl.BlockSpec(s, lambda i: tuple(0 for _ in s))
    return pl.pallas_call(
        _final_body,
        grid=(nblk,),
        in_specs=[
            pl.BlockSpec((nb, _H), lambda i: (i, 0)),
            pl.BlockSpec((nb, _H), lambda i: (i, 0)),
            pl.BlockSpec((nb, _H), lambda i: (i, 0)),
            pl.BlockSpec((1, 1, nb), lambda i: (i, 0, 0)),
            full((_H, _H)), full((1, _H)), full((_H, _DO)), full((1, _DO)),
        ],
        out_specs=[
            pl.BlockSpec((nb, _H), lambda i: (i, 0)),
            pl.BlockSpec((_G, _DO), lambda i: (0, 0)),
        ],
        out_shape=[
            jax.ShapeDtypeStruct((_N, _H), jnp.float32),
            jax.ShapeDtypeStruct((_G, _DO), jnp.float32),
        ],
        scratch_shapes=[pltpu.VMEM((_G, _H), jnp.float32)],
        compiler_params=pltpu.CompilerParams(
            dimension_semantics=("arbitrary",)),
    )(p0, p1, h, batch3, r, b, fcw, fcb)


@functools.cache
def _make_sc_kernels():
    mesh = plsc.VectorSubcoreMesh(
        core_axis_name="c", subcore_axis_name="s",
        num_cores=_NC, num_subcores=_NS)

    @functools.partial(
        pl.kernel,
        out_type=jax.ShapeDtypeStruct((_E_PAD, _H), jnp.float32),
        mesh=mesh,
        scratch_types=[
            pltpu.VMEM((_NCHUNK, _CH), jnp.int32),
            pltpu.VMEM((_EPW, _H), jnp.float32),
            pltpu.SemaphoreType.DMA,
        ],
        compiler_params=pltpu.CompilerParams(use_tc_tiling_on_sc=False),
    )
    def _sc_gather(h_hbm, src_hbm, out_hbm, idx_v, rows_v, sem):
        c = lax.axis_index("c")
        s = lax.axis_index("s")
        wid = s * _NC + c
        pltpu.sync_copy(src_hbm.at[pl.ds(wid * _NCHUNK, _NCHUNK)], idx_v)

        def body(o, carry):
            descs = []
            for j in range(8):
                ch = o * 8 + j
                d = pltpu.async_copy(
                    h_hbm.at[idx_v.at[ch]],
                    rows_v.at[pl.ds(ch * _CH, _CH)],
                    sem)
                descs.append(d)
            for d in descs:
                d.wait()
            return carry

        lax.fori_loop(0, _NCHUNK // 8, body, 0)
        pltpu.sync_copy(rows_v, out_hbm.at[pl.ds(wid * _EPW, _EPW)])


    @functools.partial(
        pl.kernel,
        out_type=jax.ShapeDtypeStruct((_NC * _N_PAD, _H), jnp.float32),
        mesh=mesh,
        scratch_types=[
            pltpu.VMEM((_NCHUNK, _CH), jnp.int32),
            pltpu.VMEM((_EPW, _H), jnp.float32),
            pltpu.VMEM((_RPS, _H), jnp.float32),
            pltpu.VMEM_SHARED((_N_PAD, _H), jnp.float32),
            pltpu.SemaphoreType.DMA,
        ],
        compiler_params=pltpu.CompilerParams(use_tc_tiling_on_sc=False),
    )
    def _sc_scatter(msg_hbm, dst_hbm, out_hbm, idx_v, rows_v, zero_v, agg_sh, sem):
        c = lax.axis_index("c")
        s = lax.axis_index("s")
        wid = s * _NC + c

        def zbody(i, carry):
            zero_v[i, :] = jnp.zeros((_H,), jnp.float32)
            return carry

        lax.fori_loop(0, _RPS, zbody, 0)
        pltpu.sync_copy(zero_v, agg_sh.at[pl.ds(s * _RPS, _RPS)])
        plsc.subcore_barrier()

        pltpu.sync_copy(dst_hbm.at[pl.ds(wid * _NCHUNK, _NCHUNK)], idx_v)
        pltpu.sync_copy(msg_hbm.at[pl.ds(wid * _EPW, _EPW)], rows_v)

        def body(o, carry):
            descs = []
            for j in range(8):
                ch = o * 8 + j
                d = pltpu.async_copy(
                    rows_v.at[pl.ds(ch * _CH, _CH)],
                    agg_sh.at[idx_v.at[ch]],
                    sem, add=True)
                descs.append(d)
            for d in descs:
                d.wait()
            return carry

        lax.fori_loop(0, _NCHUNK // 8, body, 0)
        plsc.subcore_barrier()
        pltpu.sync_copy(agg_sh.at[pl.ds(s * _RPS, _RPS)],
                        out_hbm.at[pl.ds(c * _N_PAD + s * _RPS, _RPS)])

    return _sc_gather, _sc_scatter


def kernel(x, edge_index, edge_attr, batch, nfc_w, nfc_b, efc1_w1, efc1_b1,
           efc1_w2, efc1_b2, gc1_root, gc1_bias, efc2_w1, efc2_b1, efc2_w2,
           efc2_b2, gc2_root, gc2_bias, fc_w, fc_b):
    npad = _E_PAD - _E
    src2d = jnp.concatenate(
        [edge_index[0], jnp.zeros((npad,), jnp.int32)]).reshape(-1, _CH)
    dst2d = jnp.concatenate(
        [edge_index[1], jnp.full((npad,), _N, jnp.int32)]).reshape(-1, _CH)
    ea_p = jnp.concatenate(
        [edge_attr, jnp.zeros((npad, _DE), jnp.float32)])
    batch3 = batch.reshape(10, 1, 1000)

    p_op = jnp.asarray(np.repeat(np.eye(_H, dtype=np.float32), _H, axis=1))
    q_op = jnp.asarray(np.tile(np.eye(_H, dtype=np.float32), (1, _H)))

    v1 = efc1_w2.reshape(_H * _H, _H)
    b2m1 = efc1_b2.reshape(_H, _H)
    v2 = efc2_w2.reshape(_H * _H, _H)
    b2m2 = efc2_b2.reshape(_H, _H)

    _sc_gather, _sc_scatter = _make_sc_kernels()

    h0 = _h0(x, nfc_w, nfc_b.reshape(1, _H))

    hs1 = _sc_gather(h0, src2d)
    m1 = _msg(ea_p, hs1, efc1_w1, efc1_b1.reshape(1, _H), v1, b2m1, p_op, q_op)
    agg1 = _sc_scatter(m1, dst2d).reshape(_NC, _N_PAD, _H)
    h1 = _upd(agg1[0, :_N], agg1[1, :_N], h0, gc1_root,
              gc1_bias.reshape(1, _H))

    hs2 = _sc_gather(h1, src2d)
    m2 = _msg(ea_p, hs2, efc2_w1, efc2_b1.reshape(1, _H), v2, b2m2, p_op, q_op)
    agg2 = _sc_scatter(m2, dst2d).reshape(_NC, _N_PAD, _H)

    atom_embs, out = _final(agg2[0, :_N], agg2[1, :_N], h1, batch3, gc2_root,
                            gc2_bias.reshape(1, _H), fc_w,
                            fc_b.reshape(1, _DO))
    return (out, atom_embs)


# speedup vs baseline: 3.8051x; 1.0663x over previous
import functools
import jax
import jax.numpy as jnp
import numpy as np
from jax import lax
from jax.experimental import pallas as pl
from jax.experimental.pallas import tpu as pltpu
from jax.experimental.pallas import tpu_sc as plsc

_N, _E, _DN, _DE, _H, _DO, _G = 10000, 160000, 128, 16, 16, 64, 256

_NC, _NS = 2, 16
_NW = _NC * _NS
_CH = 128
_EPW = 5120
_E_PAD = _EPW * _NW
_NCHUNK = _EPW // _CH
_N_PAD = 10240
_RPS = _N_PAD // _NS


def _lrelu(v):
    return jnp.where(v >= 0, v, 0.01 * v)


def _h0_body(x_ref, w_ref, b_ref, o_ref):
    t = jnp.dot(x_ref[...], w_ref[...], preferred_element_type=jnp.float32)
    o_ref[...] = _lrelu(t + b_ref[...])


def _h0(x, w, b):
    return pl.pallas_call(
        _h0_body,
        out_shape=jax.ShapeDtypeStruct((_N, _H), jnp.float32),
    )(x, w, b)


def _msg_body(ea_ref, hs_ref, w1_ref, b1_ref, v_ref, b2m_ref, p_ref, q_ref,
              o_ref):
    a = jnp.maximum(
        jnp.dot(ea_ref[...], w1_ref[...], preferred_element_type=jnp.float32)
        + b1_ref[...], 0.0)
    bt = (jnp.dot(a, p_ref[...], preferred_element_type=jnp.float32)
          * jnp.dot(hs_ref[...], q_ref[...], preferred_element_type=jnp.float32))
    o_ref[...] = (
        jnp.dot(bt, v_ref[...], preferred_element_type=jnp.float32)
        + jnp.dot(hs_ref[...], b2m_ref[...], preferred_element_type=jnp.float32))


def _msg(ea_p, hs, w1, b1, v, b2m, p, q, eb=4096):
    nblk = _E_PAD // eb
    full = lambda s: pl.BlockSpec(s, lambda i: (0, 0))
    return pl.pallas_call(
        _msg_body,
        grid=(nblk,),
        in_specs=[
            pl.BlockSpec((eb, _DE), lambda i: (i, 0)),
            pl.BlockSpec((eb, _H), lambda i: (i, 0)),
            full((_DE, _H)), full((1, _H)), full((_H * _H, _H)),
            full((_H, _H)), full((_H, _H * _H)), full((_H, _H * _H)),
        ],
        out_specs=pl.BlockSpec((eb, _H), lambda i: (i, 0)),
        out_shape=jax.ShapeDtypeStruct((_E_PAD, _H), jnp.float32),
        compiler_params=pltpu.CompilerParams(
            dimension_semantics=("arbitrary",)),
    )(ea_p, hs, w1, b1, v, b2m, p, q)


def _upd_body(p0_ref, p1_ref, h_ref, r_ref, b_ref, o_ref):
    t = (p0_ref[...] + p1_ref[...]
         + jnp.dot(h_ref[...], r_ref[...], preferred_element_type=jnp.float32)
         + b_ref[...])
    o_ref[...] = _lrelu(t)


def _upd(p0, p1, h, r, b, nb=2000):
    nblk = _N // nb
    full = lambda s: pl.BlockSpec(s, lambda i: (0, 0))
    return pl.pallas_call(
        _upd_body,
        grid=(nblk,),
        in_specs=[
            pl.BlockSpec((nb, _H), lambda i: (i, 0)),
            pl.BlockSpec((nb, _H), lambda i: (i, 0)),
            pl.BlockSpec((nb, _H), lambda i: (i, 0)),
            full((_H, _H)), full((1, _H)),
        ],
        out_specs=pl.BlockSpec((nb, _H), lambda i: (i, 0)),
        out_shape=jax.ShapeDtypeStruct((_N, _H), jnp.float32),
        compiler_params=pltpu.CompilerParams(
            dimension_semantics=("parallel",)),
    )(p0, p1, h, r, b)


def _final_body(p0_ref, p1_ref, h_ref, batch_ref, r_ref, b_ref, fcw_ref,
                fcb_ref, ae_ref, out_ref, acc_ref):
    i = pl.program_id(0)
    nblk = pl.num_programs(0)
    t = (p0_ref[...] + p1_ref[...]
         + jnp.dot(h_ref[...], r_ref[...], preferred_element_type=jnp.float32)
         + b_ref[...])
    ae = _lrelu(t)
    ae_ref[...] = ae
    bvec = batch_ref[0, 0, :]
    onehot_t = (lax.broadcasted_iota(jnp.int32, (_G, bvec.shape[0]), 0)
                == bvec[None, :]).astype(jnp.float32)
    part = jnp.dot(onehot_t, ae, preferred_element_type=jnp.float32)

    @pl.when(i == 0)
    def _():
        acc_ref[...] = part

    @pl.when(i > 0)
    def _():
        acc_ref[...] = acc_ref[...] + part

    @pl.when(i == nblk - 1)
    def _():
        pooled = acc_ref[...]
        nrm = jnp.sqrt(jnp.sum(pooled * pooled, axis=1, keepdims=True))
        normed = pooled / jnp.maximum(nrm, 1e-12)
        out_ref[...] = (
            jnp.dot(normed, fcw_ref[...], preferred_element_type=jnp.float32)
            + fcb_ref[...])


def _final(p0, p1, h, batch3, r, b, fcw, fcb, nb=1000):
    nblk = _N // nb
    full = lambda s: pl.BlockSpec(s, lambda i: tuple(0 for _ in s))
    return pl.pallas_call(
        _final_body,
        grid=(nblk,),
        in_specs=[
            pl.BlockSpec((nb, _H), lambda i: (i, 0)),
            pl.BlockSpec((nb, _H), lambda i: (i, 0)),
            pl.BlockSpec((nb, _H), lambda i: (i, 0)),
            pl.BlockSpec((1, 1, nb), lambda i: (i, 0, 0)),
            full((_H, _H)), full((1, _H)), full((_H, _DO)), full((1, _DO)),
        ],
        out_specs=[
            pl.BlockSpec((nb, _H), lambda i: (i, 0)),
            pl.BlockSpec((_G, _DO), lambda i: (0, 0)),
        ],
        out_shape=[
            jax.ShapeDtypeStruct((_N, _H), jnp.float32),
            jax.ShapeDtypeStruct((_G, _DO), jnp.float32),
        ],
        scratch_shapes=[pltpu.VMEM((_G, _H), jnp.float32)],
        compiler_params=pltpu.CompilerParams(
            dimension_semantics=("arbitrary",)),
    )(p0, p1, h, batch3, r, b, fcw, fcb)


@functools.cache
def _make_sc_kernels():
    mesh = plsc.VectorSubcoreMesh(
        core_axis_name="c", subcore_axis_name="s",
        num_cores=_NC, num_subcores=_NS)

    @functools.partial(
        pl.kernel,
        out_type=jax.ShapeDtypeStruct((_E_PAD, _H), jnp.float32),
        mesh=mesh,
        scratch_types=[
            pltpu.VMEM((_EPW,), jnp.int32),
            pltpu.VMEM((_EPW, _H), jnp.float32),
            pltpu.SemaphoreType.DMA,
        ],
        compiler_params=pltpu.CompilerParams(use_tc_tiling_on_sc=False),
    )
    def _sc_gather(h_hbm, src_hbm, out_hbm, idx_v, rows_v, sem):
        c = lax.axis_index("c")
        s = lax.axis_index("s")
        wid = s * _NC + c
        pltpu.sync_copy(src_hbm.at[pl.ds(wid * _EPW, _EPW)], idx_v)
        pltpu.async_copy(h_hbm.at[idx_v], rows_v, sem).wait()
        pltpu.sync_copy(rows_v, out_hbm.at[pl.ds(wid * _EPW, _EPW)])


    @functools.partial(
        pl.kernel,
        out_type=(jax.ShapeDtypeStruct((_N_PAD, _H), jnp.float32),
                  jax.ShapeDtypeStruct((_N_PAD, _H), jnp.float32)),
        mesh=mesh,
        scratch_types=[
            pltpu.VMEM((_EPW,), jnp.int32),
            pltpu.VMEM((_EPW, _H), jnp.float32),
            pltpu.VMEM((_RPS, _H), jnp.float32),
            pltpu.VMEM_SHARED((_N_PAD, _H), jnp.float32),
            pltpu.SemaphoreType.DMA,
        ],
        compiler_params=pltpu.CompilerParams(use_tc_tiling_on_sc=False),
    )
    def _sc_scatter(msg_hbm, dst_hbm, out0_hbm, out1_hbm, idx_v, rows_v,
                    zero_v, agg_sh, sem):
        c = lax.axis_index("c")
        s = lax.axis_index("s")
        wid = s * _NC + c

        def zbody(i, carry):
            zero_v[i, :] = jnp.zeros((_H,), jnp.float32)
            return carry

        lax.fori_loop(0, _RPS, zbody, 0)
        pltpu.sync_copy(zero_v, agg_sh.at[pl.ds(s * _RPS, _RPS)])
        plsc.subcore_barrier()

        pltpu.sync_copy(dst_hbm.at[pl.ds(wid * _EPW, _EPW)], idx_v)
        pltpu.sync_copy(msg_hbm.at[pl.ds(wid * _EPW, _EPW)], rows_v)
        pltpu.async_copy(rows_v, agg_sh.at[idx_v], sem, add=True).wait()
        plsc.subcore_barrier()

        @pl.when(c == 0)
        def _():
            pltpu.sync_copy(agg_sh.at[pl.ds(s * _RPS, _RPS)],
                            out0_hbm.at[pl.ds(s * _RPS, _RPS)])

        @pl.when(c == 1)
        def _():
            pltpu.sync_copy(agg_sh.at[pl.ds(s * _RPS, _RPS)],
                            out1_hbm.at[pl.ds(s * _RPS, _RPS)])

    return _sc_gather, _sc_scatter


def kernel(x, edge_index, edge_attr, batch, nfc_w, nfc_b, efc1_w1, efc1_b1,
           efc1_w2, efc1_b2, gc1_root, gc1_bias, efc2_w1, efc2_b1, efc2_w2,
           efc2_b2, gc2_root, gc2_bias, fc_w, fc_b):
    npad = _E_PAD - _E
    src_p = jnp.concatenate(
        [edge_index[0], jnp.zeros((npad,), jnp.int32)])
    dst_p = jnp.concatenate(
        [edge_index[1], jnp.full((npad,), _N, jnp.int32)])
    batch3 = batch.reshape(10, 1, 1000)

    p_op = jnp.asarray(np.repeat(np.eye(_H, dtype=np.float32), _H, axis=1))
    q_op = jnp.asarray(np.tile(np.eye(_H, dtype=np.float32), (1, _H)))

    v1 = efc1_w2.reshape(_H * _H, _H)
    b2m1 = efc1_b2.reshape(_H, _H)
    v2 = efc2_w2.reshape(_H * _H, _H)
    b2m2 = efc2_b2.reshape(_H, _H)

    _sc_gather, _sc_scatter = _make_sc_kernels()

    h0 = _h0(x, nfc_w, nfc_b.reshape(1, _H))

    hs1 = _sc_gather(h0, src_p)
    m1 = _msg(edge_attr, hs1, efc1_w1, efc1_b1.reshape(1, _H), v1, b2m1,
              p_op, q_op)
    a1_0, a1_1 = _sc_scatter(m1, dst_p)
    h1 = _upd(a1_0, a1_1, h0, gc1_root, gc1_bias.reshape(1, _H))

    hs2 = _sc_gather(h1, src_p)
    m2 = _msg(edge_attr, hs2, efc2_w1, efc2_b1.reshape(1, _H), v2, b2m2,
              p_op, q_op)
    a2_0, a2_1 = _sc_scatter(m2, dst_p)

    atom_embs, out = _final(a2_0, a2_1, h1, batch3, gc2_root,
                            gc2_bias.reshape(1, _H), fc_w,
                            fc_b.reshape(1, _DO))
    return (out, atom_embs)


# speedup vs baseline: 5.7139x; 1.5016x over previous
import functools
import jax
import jax.numpy as jnp
import numpy as np
from jax import lax
from jax.experimental import pallas as pl
from jax.experimental.pallas import tpu as pltpu
from jax.experimental.pallas import tpu_sc as plsc

_N, _E, _DN, _DE, _H, _DO, _G = 10000, 160000, 128, 16, 16, 64, 256

_NC, _NS = 2, 16
_NW = _NC * _NS
_EPW = 5120
_E_PAD = _EPW * _NW
_N_PAD = 10240
_RPS = _N_PAD // _NS

_PK = 8 * _H
_E8 = _E_PAD // 8
_EPW8 = _EPW // 8


def _lrelu(v):
    return jnp.where(v >= 0, v, 0.01 * v)


def _h0_body(x_ref, w_ref, b_ref, o_ref):
    t = jnp.dot(x_ref[...], w_ref[...], preferred_element_type=jnp.float32)
    o_ref[...] = _lrelu(t + b_ref[...])


def _h0(x, w, b):
    return pl.pallas_call(
        _h0_body,
        out_shape=jax.ShapeDtypeStruct((_N, _H), jnp.float32),
    )(x, w, b)


def _msg_body(ea_ref, hs_ref, w1_ref, b1_ref, v_ref, b2m_ref, p_ref, q_ref,
              o_ref):
    ea_bf = ea_ref[...].astype(jnp.bfloat16)
    hs_bf = hs_ref[...].astype(jnp.bfloat16)
    a = jnp.maximum(
        jnp.dot(ea_bf, w1_ref[...], preferred_element_type=jnp.float32)
        + b1_ref[...], 0.0)
    bt = (jnp.dot(a.astype(jnp.bfloat16), p_ref[...],
                  preferred_element_type=jnp.float32)
          * jnp.dot(hs_bf, q_ref[...], preferred_element_type=jnp.float32))
    o_ref[...] = (
        jnp.dot(bt.astype(jnp.bfloat16), v_ref[...],
                preferred_element_type=jnp.float32)
        + jnp.dot(hs_bf, b2m_ref[...], preferred_element_type=jnp.float32))


def _msg(ea8, hs_p, w1bd, b1t, vbd, b2mbd, pbd, qbd, eb8=256):
    nblk = _E8 // eb8
    last_ea = (_E // 8 - 1) // eb8
    full = lambda s: pl.BlockSpec(s, lambda i: (0, 0))
    return pl.pallas_call(
        _msg_body,
        grid=(nblk,),
        in_specs=[
            pl.BlockSpec((eb8, _PK), lambda i: (jnp.minimum(i, last_ea), 0)),
            pl.BlockSpec((eb8, _PK), lambda i: (i, 0)),
            full((_PK, _PK)), full((1, _PK)), full((8 * _H * _H, _PK)),
            full((_PK, _PK)), full((_PK, 8 * _H * _H)),
            full((_PK, 8 * _H * _H)),
        ],
        out_specs=pl.BlockSpec((eb8, _PK), lambda i: (i, 0)),
        out_shape=jax.ShapeDtypeStruct((_E8, _PK), jnp.float32),
        compiler_params=pltpu.CompilerParams(
            dimension_semantics=("arbitrary",)),
    )(ea8, hs_p, w1bd, b1t, vbd, b2mbd, pbd, qbd)


def _upd_body(p0_ref, p1_ref, h_ref, r_ref, b_ref, o_ref):
    t = (p0_ref[...] + p1_ref[...]
         + jnp.dot(h_ref[...], r_ref[...], preferred_element_type=jnp.float32)
         + b_ref[...])
    o_ref[...] = _lrelu(t)


def _upd(p0, p1, h, r, b, nb=2000):
    nblk = _N // nb
    full = lambda s: pl.BlockSpec(s, lambda i: (0, 0))
    return pl.pallas_call(
        _upd_body,
        grid=(nblk,),
        in_specs=[
            pl.BlockSpec((nb, _H), lambda i: (i, 0)),
            pl.BlockSpec((nb, _H), lambda i: (i, 0)),
            pl.BlockSpec((nb, _H), lambda i: (i, 0)),
            full((_H, _H)), full((1, _H)),
        ],
        out_specs=pl.BlockSpec((nb, _H), lambda i: (i, 0)),
        out_shape=jax.ShapeDtypeStruct((_N, _H), jnp.float32),
        compiler_params=pltpu.CompilerParams(
            dimension_semantics=("parallel",)),
    )(p0, p1, h, r, b)


def _final_body(p0_ref, p1_ref, h_ref, batch_ref, r_ref, b_ref, fcw_ref,
                fcb_ref, ae_ref, out_ref, acc_ref):
    i = pl.program_id(0)
    nblk = pl.num_programs(0)
    t = (p0_ref[...] + p1_ref[...]
         + jnp.dot(h_ref[...], r_ref[...], preferred_element_type=jnp.float32)
         + b_ref[...])
    ae = _lrelu(t)
    ae_ref[...] = ae
    bvec = batch_ref[0, 0, :]
    onehot_t = (lax.broadcasted_iota(jnp.int32, (_G, bvec.shape[0]), 0)
                == bvec[None, :]).astype(jnp.float32)
    part = jnp.dot(onehot_t, ae, preferred_element_type=jnp.float32)

    @pl.when(i == 0)
    def _():
        acc_ref[...] = part

    @pl.when(i > 0)
    def _():
        acc_ref[...] = acc_ref[...] + part

    @pl.when(i == nblk - 1)
    def _():
        pooled = acc_ref[...]
        nrm = jnp.sqrt(jnp.sum(pooled * pooled, axis=1, keepdims=True))
        normed = pooled / jnp.maximum(nrm, 1e-12)
        out_ref[...] = (
            jnp.dot(normed, fcw_ref[...], preferred_element_type=jnp.float32)
            + fcb_ref[...])


def _final(p0, p1, h, batch3, r, b, fcw, fcb, nb=1000):
    nblk = _N // nb
    full = lambda s: pl.BlockSpec(s, lambda i: tuple(0 for _ in s))
    return pl.pallas_call(
        _final_body,
        grid=(nblk,),
        in_specs=[
            pl.BlockSpec((nb, _H), lambda i: (i, 0)),
            pl.BlockSpec((nb, _H), lambda i: (i, 0)),
            pl.BlockSpec((nb, _H), lambda i: (i, 0)),
            pl.BlockSpec((1, 1, nb), lambda i: (i, 0, 0)),
            full((_H, _H)), full((1, _H)), full((_H, _DO)), full((1, _DO)),
        ],
        out_specs=[
            pl.BlockSpec((nb, _H), lambda i: (i, 0)),
            pl.BlockSpec((_G, _DO), lambda i: (0, 0)),
        ],
        out_shape=[
            jax.ShapeDtypeStruct((_N, _H), jnp.float32),
            jax.ShapeDtypeStruct((_G, _DO), jnp.float32),
        ],
        scratch_shapes=[pltpu.VMEM((_G, _H), jnp.float32)],
        compiler_params=pltpu.CompilerParams(
            dimension_semantics=("arbitrary",)),
    )(p0, p1, h, batch3, r, b, fcw, fcb)


@functools.cache
def _make_sc_kernels():
    mesh = plsc.VectorSubcoreMesh(
        core_axis_name="c", subcore_axis_name="s",
        num_cores=_NC, num_subcores=_NS)

    @functools.partial(
        pl.kernel,
        out_type=jax.ShapeDtypeStruct((_E_PAD, _H), jnp.float32),
        mesh=mesh,
        scratch_types=[
            pltpu.VMEM((_EPW,), jnp.int32),
            pltpu.VMEM((_EPW, _H), jnp.float32),
            pltpu.SemaphoreType.DMA,
        ],
        compiler_params=pltpu.CompilerParams(use_tc_tiling_on_sc=False),
    )
    def _sc_gather(h_hbm, src_hbm, out_hbm, idx_v, rows_v, sem):
        c = lax.axis_index("c")
        s = lax.axis_index("s")
        wid = s * _NC + c
        pltpu.sync_copy(src_hbm.at[pl.ds(wid * _EPW, _EPW)], idx_v)
        pltpu.async_copy(h_hbm.at[idx_v], rows_v, sem).wait()
        pltpu.sync_copy(rows_v, out_hbm.at[pl.ds(wid * _EPW, _EPW)])


    @functools.partial(
        pl.kernel,
        out_type=(jax.ShapeDtypeStruct((_N_PAD, _H), jnp.float32),
                  jax.ShapeDtypeStruct((_N_PAD, _H), jnp.float32)),
        mesh=mesh,
        scratch_types=[
            pltpu.VMEM((_EPW,), jnp.int32),
            pltpu.VMEM((_EPW, _H), jnp.float32),
            pltpu.VMEM((_RPS, _H), jnp.float32),
            pltpu.VMEM_SHARED((_N_PAD, _H), jnp.float32),
            pltpu.SemaphoreType.DMA,
        ],
        compiler_params=pltpu.CompilerParams(use_tc_tiling_on_sc=False),
    )
    def _sc_scatter(msg_hbm, dst_hbm, out0_hbm, out1_hbm, idx_v, rows_v,
                    zero_v, agg_sh, sem):
        c = lax.axis_index("c")
        s = lax.axis_index("s")
        wid = s * _NC + c

        def zbody(i, carry):
            zero_v[i, :] = jnp.zeros((_H,), jnp.float32)
            return carry

        lax.fori_loop(0, _RPS, zbody, 0)
        pltpu.sync_copy(zero_v, agg_sh.at[pl.ds(s * _RPS, _RPS)])
        plsc.subcore_barrier()

        pltpu.sync_copy(dst_hbm.at[pl.ds(wid * _EPW, _EPW)], idx_v)
        pltpu.sync_copy(msg_hbm.at[pl.ds(wid * _EPW, _EPW)], rows_v)
        pltpu.async_copy(rows_v, agg_sh.at[idx_v], sem, add=True).wait()
        plsc.subcore_barrier()

        @pl.when(c == 0)
        def _():
            pltpu.sync_copy(agg_sh.at[pl.ds(s * _RPS, _RPS)],
                            out0_hbm.at[pl.ds(s * _RPS, _RPS)])

        @pl.when(c == 1)
        def _():
            pltpu.sync_copy(agg_sh.at[pl.ds(s * _RPS, _RPS)],
                            out1_hbm.at[pl.ds(s * _RPS, _RPS)])

    return _sc_gather, _sc_scatter


def kernel(x, edge_index, edge_attr, batch, nfc_w, nfc_b, efc1_w1, efc1_b1,
           efc1_w2, efc1_b2, gc1_root, gc1_bias, efc2_w1, efc2_b1, efc2_w2,
           efc2_b2, gc2_root, gc2_bias, fc_w, fc_b):
    npad = _E_PAD - _E
    src_p = jnp.concatenate(
        [edge_index[0], jnp.zeros((npad,), jnp.int32)])
    dst_p = jnp.concatenate(
        [edge_index[1], jnp.full((npad,), _N, jnp.int32)])
    batch3 = batch.reshape(10, 1, 1000)

    eye8 = jnp.eye(8, dtype=jnp.float32)
    p_op = jnp.asarray(np.repeat(np.eye(_H, dtype=np.float32), _H, axis=1))
    q_op = jnp.asarray(np.tile(np.eye(_H, dtype=np.float32), (1, _H)))
    pbd = jnp.kron(eye8, p_op).astype(jnp.bfloat16)
    qbd = jnp.kron(eye8, q_op).astype(jnp.bfloat16)

    def bd(m):
        return jnp.kron(eye8, m).astype(jnp.bfloat16)

    w1bd1 = bd(efc1_w1)
    w1bd2 = bd(efc2_w1)
    vbd1 = bd(efc1_w2.reshape(_H * _H, _H))
    vbd2 = bd(efc2_w2.reshape(_H * _H, _H))
    b2mbd1 = bd(efc1_b2.reshape(_H, _H))
    b2mbd2 = bd(efc2_b2.reshape(_H, _H))
    b1t1 = jnp.tile(efc1_b1, 8).reshape(1, _PK)
    b1t2 = jnp.tile(efc2_b1, 8).reshape(1, _PK)

    ea8 = edge_attr.reshape(_E // 8, _PK)

    _sc_gather, _sc_scatter = _make_sc_kernels()

    h0 = _h0(x, nfc_w, nfc_b.reshape(1, _H))

    hs1 = _sc_gather(h0, src_p).reshape(_E8, _PK)
    m1 = _msg(ea8, hs1, w1bd1, b1t1, vbd1, b2mbd1, pbd, qbd)
    a1_0, a1_1 = _sc_scatter(m1.reshape(_E_PAD, _H), dst_p)
    h1 = _upd(a1_0, a1_1, h0, gc1_root, gc1_bias.reshape(1, _H))

    hs2 = _sc_gather(h1, src_p).reshape(_E8, _PK)
    m2 = _msg(ea8, hs2, w1bd2, b1t2, vbd2, b2mbd2, pbd, qbd)
    a2_0, a2_1 = _sc_scatter(m2.reshape(_E_PAD, _H), dst_p)

    atom_embs, out = _final(a2_0, a2_1, h1, batch3, gc2_root,
                            gc2_bias.reshape(1, _H), fc_w,
                            fc_b.reshape(1, _DO))
    return (out, atom_embs)
